# baseline (speedup 1.0000x reference)
import functools

import jax
import jax.numpy as jnp
from jax import lax
from jax.experimental import pallas as pl
from jax.experimental.pallas import tpu as pltpu
from jax.experimental.pallas import tpu_sc as plsc

VOCAB = 100000
EMB = 128
TEMPLATES = 1000
BATCH = 4096
SEQ = 50

NC = 2
NS = 16
NW = NC * NS
LANES = 16
NCHUNK = 8

ELEMS_PER_CHUNK = 4
ROWS_PER_CHUNK = ELEMS_PER_CHUNK * SEQ
SPLIT = 128
NBUF = 4

_MESH = plsc.VectorSubcoreMesh(
    core_axis_name="c", subcore_axis_name="s", num_cores=NC, num_subcores=NS)


def _make_pool(batch):
    b_per_w = batch // NW
    chunks = b_per_w // ELEMS_PER_CHUNK
    assert chunks % NBUF == 0

    @functools.partial(
        pl.kernel,
        out_type=jax.ShapeDtypeStruct((batch, EMB), jnp.float32),
        mesh=_MESH,
        scratch_types=[
            pltpu.VMEM((b_per_w * SEQ,), jnp.int32),
            [pltpu.VMEM((ROWS_PER_CHUNK, EMB), jnp.float32)
             for _ in range(NBUF)],
            pltpu.VMEM((b_per_w, EMB), jnp.float32),
            [pltpu.SemaphoreType.DMA for _ in range(NBUF)],
        ],
    )
    def pool(ids_hbm, table_hbm, out_hbm, idx_v, rows_bufs, stage_v, sems):
        wid = lax.axis_index("s") * NC + lax.axis_index("c")
        base = pl.multiple_of(wid * (b_per_w * SEQ), 8)
        pltpu.sync_copy(ids_hbm.at[pl.ds(base, b_per_w * SEQ)], idx_v)

        def issue(c, buf):
            off = pl.multiple_of(c * ROWS_PER_CHUNK, 8)
            pltpu.make_async_copy(
                table_hbm.at[idx_v.at[pl.ds(off, SPLIT)]],
                rows_bufs[buf].at[pl.ds(0, SPLIT)], sems[buf],
            ).start()
            off2 = pl.multiple_of(off + SPLIT, 8)
            pltpu.make_async_copy(
                table_hbm.at[idx_v.at[pl.ds(off2, ROWS_PER_CHUNK - SPLIT)]],
                rows_bufs[buf].at[pl.ds(SPLIT, ROWS_PER_CHUNK - SPLIT)],
                sems[buf],
            ).start()

        for b in range(NBUF - 1):
            issue(jnp.int32(b), b)

        def outer(g, carry):
            for b in range(NBUF):
                c = g * NBUF + b
                nxt = c + NBUF - 1

                @pl.when(nxt < chunks)
                def _():
                    issue(nxt, (b + NBUF - 1) % NBUF)

                rows_v = rows_bufs[b]
                pltpu.make_async_copy(
                    table_hbm.at[idx_v.at[pl.ds(0, ROWS_PER_CHUNK)]],
                    rows_v, sems[b],
                ).wait()
                for e in range(ELEMS_PER_CHUNK):
                    row0 = e * SEQ

                    def seq_body(s, accs, _row0=row0):
                        return tuple(
                            accs[k] + rows_v[_row0 + s,
                                             pl.ds(k * LANES, LANES)]
                            for k in range(NCHUNK)
                        )

                    accs = lax.fori_loop(
                        0, SEQ, seq_body,
                        tuple(jnp.zeros((LANES,), jnp.float32)
                              for _ in range(NCHUNK)),
                        unroll=10)
                    out_row = c * ELEMS_PER_CHUNK + e
                    for k in range(NCHUNK):
                        stage_v[out_row, pl.ds(k * LANES, LANES)] = (
                            accs[k] * jnp.float32(1.0 / SEQ))
            return carry

        lax.fori_loop(0, chunks // NBUF, outer, jnp.int32(0))
        pltpu.sync_copy(stage_v, out_hbm.at[pl.ds(wid * b_per_w, b_per_w)])

    return pool


def _linear_body(mean_ref, w_ref, b_ref, out_ref):
    out_ref[...] = (
        lax.dot_general(
            mean_ref[...], w_ref[...],
            dimension_numbers=(((1,), (1,)), ((), ())),
            preferred_element_type=jnp.float32,
        )
        + b_ref[...]
    )


_B_BLK = 512


def _linear(mean_emb, W, b2d):
    batch = mean_emb.shape[0]
    return pl.pallas_call(
        _linear_body,
        grid=(batch // _B_BLK,),
        in_specs=[
            pl.BlockSpec((_B_BLK, EMB), lambda i: (i, 0)),
            pl.BlockSpec((TEMPLATES, EMB), lambda i: (0, 0)),
            pl.BlockSpec((1, TEMPLATES), lambda i: (0, 0)),
        ],
        out_specs=pl.BlockSpec((_B_BLK, TEMPLATES), lambda i: (i, 0)),
        out_shape=jax.ShapeDtypeStruct((batch, TEMPLATES), jnp.float32),
    )(mean_emb, W, b2d)


NSPLIT = 1


def kernel(input_ids, emb_table, W, b):
    if input_ids.dtype != jnp.int32:
        input_ids = input_ids.astype(jnp.int32)
    ids_flat = input_ids.reshape(BATCH * SEQ)
    b2d = b.reshape(1, TEMPLATES)
    sub = BATCH // NSPLIT
    pool = _make_pool(sub)
    outs = []
    for i in range(NSPLIT):
        ids_i = (ids_flat if NSPLIT == 1 else
                 lax.slice(ids_flat, (i * sub * SEQ,), ((i + 1) * sub * SEQ,)))
        mean_i = pool(ids_i, emb_table)
        outs.append(_linear(mean_i, W, b2d))
    return outs[0] if NSPLIT == 1 else jnp.concatenate(outs, axis=0)


# speedup vs baseline: 1.1861x; 1.1861x over previous
import functools

import jax
import jax.numpy as jnp
from jax import lax
from jax.experimental import pallas as pl
from jax.experimental.pallas import tpu as pltpu
from jax.experimental.pallas import tpu_sc as plsc

VOCAB = 100000
EMB = 128
TEMPLATES = 1000
BATCH = 4096
SEQ = 50

NC = 2
NS = 16
NW = NC * NS
LANES = 16
NCHUNK = 8

ELEMS_PER_CHUNK = 4
ROWS_PER_CHUNK = ELEMS_PER_CHUNK * SEQ
SPLIT = 128
NBUF = 4

_MESH = plsc.VectorSubcoreMesh(
    core_axis_name="c", subcore_axis_name="s", num_cores=NC, num_subcores=NS)


def _make_pool(batch):
    b_per_w = batch // NW
    chunks = b_per_w // ELEMS_PER_CHUNK
    assert chunks % NBUF == 0

    @functools.partial(
        pl.kernel,
        out_type=jax.ShapeDtypeStruct((batch, EMB), jnp.float32),
        mesh=_MESH,
        scratch_types=[
            pltpu.VMEM((b_per_w * SEQ,), jnp.int32),
            [pltpu.VMEM((ROWS_PER_CHUNK, EMB), jnp.float32)
             for _ in range(NBUF)],
            pltpu.VMEM((b_per_w, EMB), jnp.float32),
            [pltpu.SemaphoreType.DMA for _ in range(NBUF)],
        ],
    )
    def pool(ids_hbm, table_hbm, out_hbm, idx_v, rows_bufs, stage_v, sems):
        wid = lax.axis_index("s") * NC + lax.axis_index("c")
        base = pl.multiple_of(wid * (b_per_w * SEQ), 8)
        pltpu.sync_copy(ids_hbm.at[pl.ds(base, b_per_w * SEQ)], idx_v)

        def issue(c, buf):
            off = pl.multiple_of(c * ROWS_PER_CHUNK, 8)
            pltpu.make_async_copy(
                table_hbm.at[idx_v.at[pl.ds(off, SPLIT)]],
                rows_bufs[buf].at[pl.ds(0, SPLIT)], sems[buf],
            ).start()
            off2 = pl.multiple_of(off + SPLIT, 8)
            pltpu.make_async_copy(
                table_hbm.at[idx_v.at[pl.ds(off2, ROWS_PER_CHUNK - SPLIT)]],
                rows_bufs[buf].at[pl.ds(SPLIT, ROWS_PER_CHUNK - SPLIT)],
                sems[buf],
            ).start()

        for b in range(NBUF - 1):
            issue(jnp.int32(b), b)

        def outer(g, carry):
            for b in range(NBUF):
                c = g * NBUF + b
                nxt = c + NBUF - 1

                @pl.when(nxt < chunks)
                def _():
                    issue(nxt, (b + NBUF - 1) % NBUF)

                rows_v = rows_bufs[b]
                pltpu.make_async_copy(
                    table_hbm.at[idx_v.at[pl.ds(0, ROWS_PER_CHUNK)]],
                    rows_v, sems[b],
                ).wait()
                for e in range(ELEMS_PER_CHUNK):
                    row0 = e * SEQ

                    def seq_body(s, accs, _row0=row0):
                        return tuple(
                            accs[k] + rows_v[_row0 + s,
                                             pl.ds(k * LANES, LANES)]
                            for k in range(NCHUNK)
                        )

                    accs = lax.fori_loop(
                        0, SEQ, seq_body,
                        tuple(jnp.zeros((LANES,), jnp.float32)
                              for _ in range(NCHUNK)),
                        unroll=10)
                    out_row = c * ELEMS_PER_CHUNK + e
                    for k in range(NCHUNK):
                        stage_v[out_row, pl.ds(k * LANES, LANES)] = (
                            accs[k] * jnp.float32(1.0 / SEQ))
            return carry

        lax.fori_loop(0, chunks // NBUF, outer, jnp.int32(0))
        pltpu.sync_copy(stage_v, out_hbm.at[pl.ds(wid * b_per_w, b_per_w)])

    return pool


def _linear_t_body(w_ref, mean_ref, b_ref, out_ref):
    out_ref[...] = (
        lax.dot_general(
            w_ref[...], mean_ref[...],
            dimension_numbers=(((1,), (1,)), ((), ())),
            preferred_element_type=jnp.float32,
        )
        + b_ref[...]
    )


_B_BLK = 512


def _linear_t(mean_emb, W, b2d):
    batch = mean_emb.shape[0]
    return pl.pallas_call(
        _linear_t_body,
        grid=(batch // _B_BLK,),
        in_specs=[
            pl.BlockSpec((TEMPLATES, EMB), lambda i: (0, 0)),
            pl.BlockSpec((_B_BLK, EMB), lambda i: (i, 0)),
            pl.BlockSpec((TEMPLATES, 1), lambda i: (0, 0)),
        ],
        out_specs=pl.BlockSpec((TEMPLATES, _B_BLK), lambda i: (0, i)),
        out_shape=jax.ShapeDtypeStruct((TEMPLATES, batch), jnp.float32),
    )(W, mean_emb, b2d)


NSPLIT = 1


def kernel(input_ids, emb_table, W, b):
    if input_ids.dtype != jnp.int32:
        input_ids = input_ids.astype(jnp.int32)
    ids_flat = input_ids.reshape(BATCH * SEQ)
    b2d = b.reshape(TEMPLATES, 1)
    sub = BATCH // NSPLIT
    pool = _make_pool(sub)
    outs = []
    for i in range(NSPLIT):
        ids_i = (ids_flat if NSPLIT == 1 else
                 lax.slice(ids_flat, (i * sub * SEQ,), ((i + 1) * sub * SEQ,)))
        mean_i = pool(ids_i, emb_table)
        outs.append(_linear_t(mean_i, W, b2d))
    out_t = outs[0] if NSPLIT == 1 else jnp.concatenate(outs, axis=1)
    return out_t.T


# speedup vs baseline: 1.1908x; 1.0040x over previous
import functools

import jax
import jax.numpy as jnp
from jax import lax
from jax.experimental import pallas as pl
from jax.experimental.pallas import tpu as pltpu
from jax.experimental.pallas import tpu_sc as plsc

VOCAB = 100000
EMB = 128
TEMPLATES = 1000
BATCH = 4096
SEQ = 50

NC = 2
NS = 16
NW = NC * NS
LANES = 16
NCHUNK = 8

ELEMS_PER_CHUNK = 4
ROWS_PER_CHUNK = ELEMS_PER_CHUNK * SEQ
SPLIT = 128
NBUF = 4

_MESH = plsc.VectorSubcoreMesh(
    core_axis_name="c", subcore_axis_name="s", num_cores=NC, num_subcores=NS)


def _make_pool(batch):
    b_per_w = batch // NW
    chunks = b_per_w // ELEMS_PER_CHUNK
    assert chunks % NBUF == 0

    @functools.partial(
        pl.kernel,
        out_type=jax.ShapeDtypeStruct((batch, EMB), jnp.float32),
        mesh=_MESH,
        scratch_types=[
            pltpu.VMEM((b_per_w * SEQ,), jnp.int32),
            [pltpu.VMEM((ROWS_PER_CHUNK, EMB), jnp.float32)
             for _ in range(NBUF)],
            pltpu.VMEM((b_per_w, EMB), jnp.float32),
            [pltpu.SemaphoreType.DMA for _ in range(NBUF)],
        ],
    )
    def pool(ids_hbm, table_hbm, out_hbm, idx_v, rows_bufs, stage_v, sems):
        wid = lax.axis_index("s") * NC + lax.axis_index("c")
        base = pl.multiple_of(wid * (b_per_w * SEQ), 8)
        pltpu.sync_copy(ids_hbm.at[pl.ds(base, b_per_w * SEQ)], idx_v)

        def issue(c, buf):
            off = pl.multiple_of(c * ROWS_PER_CHUNK, 8)
            pltpu.make_async_copy(
                table_hbm.at[idx_v.at[pl.ds(off, SPLIT)]],
                rows_bufs[buf].at[pl.ds(0, SPLIT)], sems[buf],
            ).start()
            off2 = pl.multiple_of(off + SPLIT, 8)
            pltpu.make_async_copy(
                table_hbm.at[idx_v.at[pl.ds(off2, ROWS_PER_CHUNK - SPLIT)]],
                rows_bufs[buf].at[pl.ds(SPLIT, ROWS_PER_CHUNK - SPLIT)],
                sems[buf],
            ).start()

        for b in range(NBUF - 1):
            issue(jnp.int32(b), b)

        def outer(g, carry):
            for b in range(NBUF):
                c = g * NBUF + b
                nxt = c + NBUF - 1

                @pl.when(nxt < chunks)
                def _():
                    issue(nxt, (b + NBUF - 1) % NBUF)

                rows_v = rows_bufs[b]
                pltpu.make_async_copy(
                    table_hbm.at[idx_v.at[pl.ds(0, ROWS_PER_CHUNK)]],
                    rows_v, sems[b],
                ).wait()
                for e in range(ELEMS_PER_CHUNK):
                    row0 = e * SEQ

                    def seq_body(s, accs, _row0=row0):
                        return tuple(
                            accs[k] + rows_v[_row0 + s,
                                             pl.ds(k * LANES, LANES)]
                            for k in range(NCHUNK)
                        )

                    accs = lax.fori_loop(
                        0, SEQ, seq_body,
                        tuple(jnp.zeros((LANES,), jnp.float32)
                              for _ in range(NCHUNK)),
                        unroll=10)
                    out_row = c * ELEMS_PER_CHUNK + e
                    for k in range(NCHUNK):
                        stage_v[out_row, pl.ds(k * LANES, LANES)] = (
                            accs[k] * jnp.float32(1.0 / SEQ))
            return carry

        lax.fori_loop(0, chunks // NBUF, outer, jnp.int32(0))
        pltpu.sync_copy(stage_v, out_hbm.at[pl.ds(wid * b_per_w, b_per_w)])

    return pool


def _linear_t_body(w_ref, mean_ref, b_ref, out_ref):
    out_ref[...] = (
        lax.dot_general(
            w_ref[...], mean_ref[...],
            dimension_numbers=(((1,), (1,)), ((), ())),
            preferred_element_type=jnp.float32,
        )
        + b_ref[...]
    )


_B_BLK = 512


def _linear_t(mean_emb, W, b2d):
    batch = mean_emb.shape[0]
    return pl.pallas_call(
        _linear_t_body,
        grid=(batch // _B_BLK,),
        in_specs=[
            pl.BlockSpec((TEMPLATES, EMB), lambda i: (0, 0)),
            pl.BlockSpec((_B_BLK, EMB), lambda i: (i, 0)),
            pl.BlockSpec((TEMPLATES, 1), lambda i: (0, 0)),
        ],
        out_specs=pl.BlockSpec((TEMPLATES, _B_BLK), lambda i: (0, i)),
        out_shape=jax.ShapeDtypeStruct((TEMPLATES, batch), jnp.float32),
    )(W, mean_emb, b2d)


def kernel(input_ids, emb_table, W, b):
    if input_ids.dtype != jnp.int32:
        input_ids = input_ids.astype(jnp.int32)
    ids_flat = input_ids.reshape(BATCH * SEQ)
    mean_emb = _make_pool(BATCH)(ids_flat, emb_table)
    return _linear_t(mean_emb, W, b.reshape(TEMPLATES, 1)).T
